# baseline (speedup 1.0000x reference)
import functools

import jax
import jax.numpy as jnp
from jax import lax
from jax.experimental import pallas as pl
from jax.experimental.pallas import tpu as pltpu
from jax.experimental.pallas import tpu_sc as plsc

ALL_VARS = 1048576
BATCH = 16384
N_NODES = 5
N_STATES = 16


_LANES = 128
_ROWS = ALL_VARS // _LANES
_GRID = 2
_BLK_ROWS = _ROWS // _GRID // 8


_CH = 8


def _half_stats(ref, m_new):
    blk = ref[...].reshape(_CH, _BLK_ROWS // (8 * _CH), 8, _LANES)
    s8 = jnp.sum(jnp.exp(blk - m_new[None, None]), axis=1)
    return jnp.sum(s8, axis=0)


def _half_max(ref):
    blk = ref[...].reshape(_CH, _BLK_ROWS // (8 * _CH), 8, _LANES)
    return jnp.max(jnp.max(blk, axis=1), axis=0)


def _lse_body(*args):
    (xa_ref, xb_ref, xc_ref, xd_ref, xe_ref, xf_ref, xg_ref, xh_ref,
     out_ref, m_vec, s_vec) = args
    refs = (xa_ref, xb_ref, xc_ref, xd_ref, xe_ref, xf_ref, xg_ref, xh_ref)
    i = pl.program_id(0)
    bm = _half_max(refs[0])
    for r in refs[1:]:
        bm = jnp.maximum(bm, _half_max(r))

    @pl.when(i == 0)
    def _init():
        m_vec[...] = jnp.full((8, _LANES), -jnp.inf, jnp.float32)
        s_vec[...] = jnp.zeros((8, _LANES), jnp.float32)

    m_old = m_vec[...]
    m_new = jnp.maximum(m_old, bm)
    s_tot = s_vec[...] * jnp.exp(m_old - m_new)
    for r in refs:
        s_tot = s_tot + _half_stats(r, m_new)
    s_vec[...] = s_tot
    m_vec[...] = m_new

    @pl.when(i == _GRID - 1)
    def _fin():
        m_fin = jnp.max(m_new)
        s_fin = jnp.sum(s_vec[...] * jnp.exp(m_vec[...] - m_fin))
        out_ref[...] = jnp.full((8, _LANES), m_fin + jnp.log(s_fin),
                                jnp.float32)


def _logsumexp(logits2d):
    return pl.pallas_call(
        _lse_body,
        grid=(_GRID,),
        in_specs=[pl.BlockSpec((_BLK_ROWS, _LANES),
                               (lambda k: (lambda i: (8 * i + k, 0)))(k))
                  for k in range(8)],
        out_specs=pl.BlockSpec((8, _LANES), lambda i: (0, 0)),
        out_shape=jax.ShapeDtypeStruct((8, _LANES), jnp.float32),
        scratch_shapes=[pltpu.VMEM((8, _LANES), jnp.float32),
                        pltpu.VMEM((8, _LANES), jnp.float32)],
    )(*([logits2d] * 8))


_NC = 2
_NS = 16
_NW = _NC * _NS
_BPW = BATCH // _NW
_NROW = _BPW // 128

_sc_mesh = plsc.VectorSubcoreMesh(core_axis_name="c", subcore_axis_name="s")


@functools.partial(
    pl.kernel,
    mesh=_sc_mesh,
    out_type=jax.ShapeDtypeStruct((_NW * _NROW, 128), jnp.float32),
    scratch_types=[
        pltpu.VMEM((N_NODES, _BPW), jnp.int32),
        pltpu.VMEM((_NROW, 128), jnp.int32),
        pltpu.VMEM((_NROW, 128), jnp.float32),
        pltpu.SemaphoreType.DMA,
    ],
)
def _sc_gather(xt_hbm, logits_hbm, out_hbm, xbuf, idxbuf, valbuf, sem):
    wid = lax.axis_index("s") * _NC + lax.axis_index("c")
    base = wid * _BPW
    pltpu.sync_copy(xt_hbm.at[:, pl.ds(base, _BPW)], xbuf)

    def _build(h, carry):
        for u in range(2):
            g = h * 2 + u
            off = g * 16
            acc = xbuf[0, pl.ds(off, 16)]
            for i in range(1, N_NODES):
                acc = acc * N_STATES + xbuf[i, pl.ds(off, 16)]
            idxbuf[g // 8, pl.ds((g % 8) * 16, 16)] = acc
        return carry

    copies = []
    for j in range(_NROW):
        lax.fori_loop(j * 4, (j + 1) * 4, _build, 0)
        copies.append(
            pltpu.async_copy(logits_hbm.at[idxbuf.at[j]], valbuf.at[j], sem))
    for cp in copies:
        cp.wait()
    pltpu.sync_copy(valbuf, out_hbm.at[pl.ds(wid * _NROW, _NROW), :])


def _combine_body(g_ref, lz_ref, o_ref):
    o_ref[...] = g_ref[...] - lz_ref[0, 0]


def _combine(gathered2d, logz):
    return pl.pallas_call(
        _combine_body,
        out_shape=jax.ShapeDtypeStruct(gathered2d.shape, jnp.float32),
    )(gathered2d, logz)


def kernel(x, logits):
    gathered = _sc_gather(x.T, logits)
    logz = _logsumexp(logits.reshape(_ROWS, _LANES))
    return _combine(gathered, logz).reshape(BATCH)


# speedup vs baseline: 1.0231x; 1.0231x over previous
import functools

import jax
import jax.numpy as jnp
from jax import lax
from jax.experimental import pallas as pl
from jax.experimental.pallas import tpu as pltpu
from jax.experimental.pallas import tpu_sc as plsc

ALL_VARS = 1048576
BATCH = 16384
N_NODES = 5
N_STATES = 16


_LANES = 128
_ROWS = ALL_VARS // _LANES
_GRID = 2
_BLK_ROWS = _ROWS // _GRID // 8


_CH = 8


def _half_stats(ref, m_new):
    blk = ref[...].reshape(_CH, _BLK_ROWS // (8 * _CH), 8, _LANES)
    s8 = jnp.sum(jnp.exp(blk - m_new[None, None]), axis=1)
    return jnp.sum(s8, axis=0)


def _half_max(ref):
    blk = ref[...].reshape(_CH, _BLK_ROWS // (8 * _CH), 8, _LANES)
    return jnp.max(jnp.max(blk, axis=1), axis=0)


def _lse_body(*args):
    (xa_ref, xb_ref, xc_ref, xd_ref, xe_ref, xf_ref, xg_ref, xh_ref,
     out_ref, m_vec, s_vec) = args
    refs = (xa_ref, xb_ref, xc_ref, xd_ref, xe_ref, xf_ref, xg_ref, xh_ref)
    i = pl.program_id(0)
    bm = _half_max(refs[0])
    for r in refs[1:]:
        bm = jnp.maximum(bm, _half_max(r))

    @pl.when(i == 0)
    def _init():
        m_vec[...] = jnp.full((8, _LANES), -jnp.inf, jnp.float32)
        s_vec[...] = jnp.zeros((8, _LANES), jnp.float32)

    m_old = m_vec[...]
    m_new = jnp.maximum(m_old, bm)
    s_tot = s_vec[...] * jnp.exp(m_old - m_new)
    for r in refs:
        s_tot = s_tot + _half_stats(r, m_new)
    s_vec[...] = s_tot
    m_vec[...] = m_new

    @pl.when(i == _GRID - 1)
    def _fin():
        m_fin = jnp.max(m_new)
        s_fin = jnp.sum(s_vec[...] * jnp.exp(m_vec[...] - m_fin))
        out_ref[...] = jnp.full((8, _LANES), m_fin + jnp.log(s_fin),
                                jnp.float32)


def _logsumexp(logits2d):
    return pl.pallas_call(
        _lse_body,
        grid=(_GRID,),
        in_specs=[pl.BlockSpec((_BLK_ROWS, _LANES),
                               (lambda k: (lambda i: (8 * i + k, 0)))(k))
                  for k in range(8)],
        out_specs=pl.BlockSpec((8, _LANES), lambda i: (0, 0)),
        out_shape=jax.ShapeDtypeStruct((8, _LANES), jnp.float32),
        scratch_shapes=[pltpu.VMEM((8, _LANES), jnp.float32),
                        pltpu.VMEM((8, _LANES), jnp.float32)],
    )(*([logits2d] * 8))


_NC = 1
_NS = 16
_NW = _NC * _NS
_BPW = BATCH // _NW
_NROW = _BPW // 128

_sc_mesh = plsc.VectorSubcoreMesh(core_axis_name="c", subcore_axis_name="s",
                                  num_cores=1)


@functools.partial(
    pl.kernel,
    mesh=_sc_mesh,
    out_type=jax.ShapeDtypeStruct((_NW * _NROW, 128), jnp.float32),
    scratch_types=[
        pltpu.VMEM((N_NODES, _BPW), jnp.int32),
        pltpu.VMEM((_NROW, 128), jnp.int32),
        pltpu.VMEM((_NROW, 128), jnp.float32),
        pltpu.SemaphoreType.DMA,
    ],
)
def _sc_gather(xt_hbm, logits_hbm, out_hbm, xbuf, idxbuf, valbuf, sem):
    wid = lax.axis_index("s") * _NC + lax.axis_index("c")
    base = wid * _BPW
    pltpu.sync_copy(xt_hbm.at[:, pl.ds(base, _BPW)], xbuf)

    def _build(h, carry):
        for u in range(2):
            g = h * 2 + u
            off = g * 16
            acc = xbuf[0, pl.ds(off, 16)]
            for i in range(1, N_NODES):
                acc = acc * N_STATES + xbuf[i, pl.ds(off, 16)]
            idxbuf[g // 8, pl.ds((g % 8) * 16, 16)] = acc
        return carry

    lax.fori_loop(0, _BPW // 32, _build, 0)
    copies = [
        pltpu.async_copy(logits_hbm.at[idxbuf.at[j]], valbuf.at[j], sem)
        for j in range(_NROW)
    ]
    for cp in copies:
        cp.wait()
    pltpu.sync_copy(valbuf, out_hbm.at[pl.ds(wid * _NROW, _NROW), :])


def _combine_body(g_ref, lz_ref, o_ref):
    o_ref[...] = g_ref[...] - lz_ref[0, 0]


def _combine(gathered2d, logz):
    return pl.pallas_call(
        _combine_body,
        out_shape=jax.ShapeDtypeStruct(gathered2d.shape, jnp.float32),
    )(gathered2d, logz)


def kernel(x, logits):
    gathered = _sc_gather(x.T, logits)
    logz = _logsumexp(logits.reshape(_ROWS, _LANES))
    return _combine(gathered, logz).reshape(BATCH)
